# baseline (speedup 1.0000x reference)
import functools

import jax
import jax.numpy as jnp
from jax import lax
from jax.experimental import pallas as pl
from jax.experimental.pallas import tpu as pltpu
from jax.experimental.pallas import tpu_sc as plsc

_B, _T, _D = 1024, 50, 1000
_CW = 200
_NDC = _D // _CW
_NBT = _B // 128
_NSL = _T * _NBT
_NU = _NSL * _NDC
_NW = 32
_UPW = -(-_NU // _NW)
_NK = -(-_UPW // 2)
_MAXSL = 14


@functools.partial(
    pl.kernel,
    out_type=jax.ShapeDtypeStruct((_T, _D // 8, _NBT, 8, 128), jnp.float32),
    mesh=plsc.VectorSubcoreMesh(core_axis_name="c", subcore_axis_name="s"),
    scratch_types=[
        pltpu.VMEM((_MAXSL, 128), jnp.int32),
        pltpu.VMEM((128,), jnp.int32),
        pltpu.VMEM((128,), jnp.int32),
        pltpu.VMEM((128, _CW), jnp.float32),
        pltpu.VMEM((128, _CW), jnp.float32),
        pltpu.VMEM((_CW // 8, 8, 128), jnp.float32),
        pltpu.VMEM((_CW // 8, 8, 128), jnp.float32),
        pltpu.SemaphoreType.DMA,
        pltpu.SemaphoreType.DMA,
        pltpu.SemaphoreType.DMA,
        pltpu.SemaphoreType.DMA,
    ],
    compiler_params=pltpu.CompilerParams(
        use_tc_tiling_on_sc=False, needs_layout_passes=False),
)
def _gather_t_sc(x5_hbm, table5_hbm, out5_hbm, xslab_v, idx0, idx1,
                 gbuf0, gbuf1, tbuf0, tbuf1, gs0, gs1, ss0, ss1):
    wid = lax.axis_index("s") * 2 + lax.axis_index("c")
    base = wid * _UPW
    uend = jnp.minimum(base + _UPW, _NU)

    slc = jnp.minimum(base // _NDC, _NSL - _MAXSL)
    pltpu.sync_copy(x5_hbm.at[pl.ds(slc, _MAXSL)], xslab_v)

    idxs = (idx0, idx1)
    gbufs = (gbuf0, gbuf1)
    tbufs = (tbuf0, tbuf1)
    gsems = (gs0, gs1)
    ssems = (ss0, ss1)

    def issue_gather(u, p):
        sl = u // _NDC
        dc = u - sl * _NDC
        row = sl - slc
        for g in range(8):
            idxs[p][pl.ds(16 * g, 16)] = (
                xslab_v[row, pl.ds(16 * g, 16)] + dc)
        pltpu.async_copy(table5_hbm.at[idxs[p]], gbufs[p], gsems[p])

    def wait_gather(p):
        pltpu.make_async_copy(
            table5_hbm.at[pl.ds(0, 128)], gbufs[p], gsems[p]).wait()

    def wait_store(p):
        pltpu.make_async_copy(
            tbufs[p], out5_hbm.at[0, pl.ds(0, _CW // 8), 0], ssems[p]).wait()

    rowvs = [lax.iota(jnp.int32, 16) + 16 * g for g in range(8)]

    def transpose(p):
        def body(dtl, carry):
            for batch in range(2):
                vals = []
                for di in range(4 * batch, 4 * batch + 4):
                    col = jnp.full((16,), dtl * 8 + di, dtype=jnp.int32)
                    for g in range(8):
                        vals.append(
                            (di, g, plsc.load_gather(gbufs[p], [rowvs[g], col])))
                for di, g, v in vals:
                    tbufs[p][dtl, di, pl.ds(16 * g, 16)] = v
            return carry
        lax.fori_loop(0, _CW // 8, body, 0, unroll=2)

    def issue_store(u, p):
        sl = u // _NDC
        dc = u - sl * _NDC
        t = sl // _NBT
        bt = sl - t * _NBT
        pltpu.async_copy(
            tbufs[p], out5_hbm.at[t, pl.ds(dc * (_CW // 8), _CW // 8), bt],
            ssems[p])

    @pl.when(base < uend)
    def _():
        issue_gather(base, 0)

    @pl.when(base + 1 < uend)
    def _():
        issue_gather(base + 1, 1)

    def round_body(k, carry):
        for p in range(2):
            u = base + 2 * k + p

            @pl.when(u < uend)
            def _():
                @pl.when(k > 0)
                def _():
                    wait_store(p)
                wait_gather(p)
                transpose(p)
                issue_store(u, p)

                @pl.when(u + 2 < uend)
                def _():
                    issue_gather(u + 2, p)
        return carry

    lax.fori_loop(0, _NK, round_body, 0)

    for p in range(2):
        @pl.when(base + p < uend)
        def _():
            wait_store(p)


def kernel(x, embedding):
    x5 = (x.astype(jnp.int32).T * _NDC).reshape(_NSL, 128)
    table5 = embedding.reshape(_D * _NDC, _CW)
    p5 = _gather_t_sc(x5, table5)
    return p5.transpose(2, 4, 0, 1, 3).reshape(_B, _T, _D)


# speedup vs baseline: 1.0395x; 1.0395x over previous
import functools

import jax
import jax.numpy as jnp
from jax import lax
from jax.experimental import pallas as pl
from jax.experimental.pallas import tpu as pltpu
from jax.experimental.pallas import tpu_sc as plsc

_B, _T, _D = 1024, 50, 1000
_CW = 200
_NDC = _D // _CW
_NBT = _B // 128
_NSL = _T * _NBT
_NU = _NSL * _NDC
_NW = 32
_UPW = -(-_NU // _NW)
_NK = -(-_UPW // 2)
_MAXSL = 14


@functools.partial(
    pl.kernel,
    out_type=jax.ShapeDtypeStruct((_T, _D // 8, _NBT, 8, 128), jnp.float32),
    mesh=plsc.VectorSubcoreMesh(core_axis_name="c", subcore_axis_name="s"),
    scratch_types=[
        pltpu.VMEM((_MAXSL, 128), jnp.int32),
        pltpu.VMEM((128,), jnp.int32),
        pltpu.VMEM((128,), jnp.int32),
        pltpu.VMEM((128, _CW), jnp.float32),
        pltpu.VMEM((128, _CW), jnp.float32),
        pltpu.VMEM((_CW // 8, 8, 128), jnp.float32),
        pltpu.VMEM((_CW // 8, 8, 128), jnp.float32),
        pltpu.SemaphoreType.DMA,
        pltpu.SemaphoreType.DMA,
        pltpu.SemaphoreType.DMA,
        pltpu.SemaphoreType.DMA,
    ],
    compiler_params=pltpu.CompilerParams(
        use_tc_tiling_on_sc=False, needs_layout_passes=False),
)
def _gather_t_sc(x5_hbm, table5_hbm, out5_hbm, xslab_v, idx0, idx1,
                 gbuf0, gbuf1, tbuf0, tbuf1, gs0, gs1, ss0, ss1):
    wid = lax.axis_index("s") * 2 + lax.axis_index("c")
    base = wid * _UPW
    uend = jnp.minimum(base + _UPW, _NU)

    slc = jnp.minimum(base // _NDC, _NSL - _MAXSL)
    pltpu.sync_copy(x5_hbm.at[pl.ds(slc, _MAXSL)], xslab_v)

    idxs = (idx0, idx1)
    gbufs = (gbuf0, gbuf1)
    tbufs = (tbuf0, tbuf1)
    gsems = (gs0, gs1)
    ssems = (ss0, ss1)

    def issue_gather(u, p):
        sl = u // _NDC
        dc = u - sl * _NDC
        row = sl - slc
        for g in range(8):
            idxs[p][pl.ds(16 * g, 16)] = (
                xslab_v[row, pl.ds(16 * g, 16)] + dc)
        pltpu.async_copy(table5_hbm.at[idxs[p]], gbufs[p], gsems[p])

    def wait_gather(p):
        pltpu.make_async_copy(
            table5_hbm.at[pl.ds(0, 128)], gbufs[p], gsems[p]).wait()

    def wait_store(p):
        pltpu.make_async_copy(
            tbufs[p], out5_hbm.at[0, pl.ds(0, _CW // 8), 0], ssems[p]).wait()

    rowvs = [lax.iota(jnp.int32, 16) + 16 * g for g in range(8)]

    def transpose(p):
        def body(dtl, carry):
            for batch in range(4):
                vals = []
                for di in range(2 * batch, 2 * batch + 2):
                    col = jnp.full((16,), dtl * 8 + di, dtype=jnp.int32)
                    for g in range(8):
                        vals.append(
                            (di, g, plsc.load_gather(gbufs[p], [rowvs[g], col])))
                for di, g, v in vals:
                    tbufs[p][dtl, di, pl.ds(16 * g, 16)] = v
            return carry
        lax.fori_loop(0, _CW // 8, body, 0)

    def issue_store(u, p):
        sl = u // _NDC
        dc = u - sl * _NDC
        t = sl // _NBT
        bt = sl - t * _NBT
        pltpu.async_copy(
            tbufs[p], out5_hbm.at[t, pl.ds(dc * (_CW // 8), _CW // 8), bt],
            ssems[p])

    @pl.when(base < uend)
    def _():
        issue_gather(base, 0)

    @pl.when(base + 1 < uend)
    def _():
        issue_gather(base + 1, 1)

    def round_body(k, carry):
        for p in range(2):
            u = base + 2 * k + p

            @pl.when(u < uend)
            def _():
                @pl.when(k > 0)
                def _():
                    wait_store(p)
                wait_gather(p)
                transpose(p)
                issue_store(u, p)

                @pl.when(u + 2 < uend)
                def _():
                    issue_gather(u + 2, p)
        return carry

    lax.fori_loop(0, _NK, round_body, 0)

    for p in range(2):
        @pl.when(base + p < uend)
        def _():
            wait_store(p)


def kernel(x, embedding):
    x5 = (x.astype(jnp.int32).T * _NDC).reshape(_NSL, 128)
    table5 = embedding.reshape(_D * _NDC, _CW)
    p5 = _gather_t_sc(x5, table5)
    return p5.transpose(2, 4, 0, 1, 3).reshape(_B, _T, _D)
